# baseline (speedup 1.0000x reference)
import jax
import jax.numpy as jnp
from jax.experimental import pallas as pl
from jax.experimental.pallas import tpu as pltpu

_NUM_CLASSES_OUT = 64
_RING = 4


def _fused_body(w1_ref, b1_ref, wl_ref, bl_ref,
                hw1_ref, hb1_ref, hw2_ref, hb2_ref, hw3_ref, hb3_ref,
                scx_ref, scadj_hbm, fcx_ref, fcadj_hbm, pool_ref,
                o_ref, ring_a, ring_b, a16a, a16b, sem,
                h_a0, h_a1, h_b0, h_b1, pooled):
    n = a16a.shape[0]
    tile = ring_a.shape[1]
    nt = n // tile
    f_pad = w1_ref.shape[1] // 2
    h_pad = w1_ref.shape[2]
    num_extra = wl_ref.shape[1]
    blk = 512 if n % 512 == 0 else tile

    def copy(adj_hbm, ring, bidx, t):
        return pltpu.make_async_copy(
            adj_hbm.at[pl.ds(t * tile, tile)],
            ring.at[t % _RING],
            sem.at[bidx, t])

    for t in range(min(_RING, nt)):
        copy(scadj_hbm, ring_a, 0, t).start()
        copy(fcadj_hbm, ring_b, 1, t).start()

    pool16 = pool_ref[...].astype(jnp.bfloat16)
    x16a = scx_ref[...].astype(jnp.bfloat16)
    x16b = fcx_ref[...].astype(jnp.bfloat16)

    def l1_tile(adj_hbm, ring, a16, x16, w1x, b1x, h, bidx, t):
        copy(adj_hbm, ring, bidx, t).wait()
        a_t = ring[t % _RING].astype(jnp.bfloat16)
        a16[pl.ds(t * tile, tile), :] = a_t
        agg = jnp.dot(a_t, x16, preferred_element_type=jnp.float32)
        z = (jnp.dot(agg.astype(jnp.bfloat16), w1x[:f_pad],
                     preferred_element_type=jnp.float32)
             + jnp.dot(x16[t * tile:(t + 1) * tile], w1x[f_pad:],
                       preferred_element_type=jnp.float32)
             + b1x)
        h[pl.ds(t * tile, tile), :] = jnp.maximum(z, 0.0).astype(jnp.bfloat16)
        if t + _RING < nt:
            copy(adj_hbm, ring, bidx, t + _RING).start()

    for t in range(nt):
        l1_tile(scadj_hbm, ring_a, a16a, x16a, w1_ref[0], b1_ref[0], h_a0, 0, t)
        l1_tile(fcadj_hbm, ring_b, a16b, x16b, w1_ref[1], b1_ref[1], h_b0, 1, t)

    def sage_block(a16, h_src, h_dst, wlb, blb, r0):
        rows = pl.ds(r0, blk)
        agg = jnp.dot(a16[rows, :], h_src[...],
                      preferred_element_type=jnp.float32)
        z = (jnp.dot(agg.astype(jnp.bfloat16), wlb[:h_pad],
                     preferred_element_type=jnp.float32)
             + jnp.dot(h_src[rows, :], wlb[h_pad:],
                       preferred_element_type=jnp.float32)
             + blb)
        h_dst[rows, :] = jnp.maximum(z, 0.0).astype(jnp.bfloat16)

    bufs_a = (h_a0, h_a1)
    bufs_b = (h_b0, h_b1)
    for layer in range(num_extra):
        sa, da = bufs_a[layer % 2], bufs_a[(layer + 1) % 2]
        sb, db = bufs_b[layer % 2], bufs_b[(layer + 1) % 2]
        for r0 in range(0, n, blk):
            sage_block(a16a, sa, da, wl_ref[0, layer], bl_ref[0, layer], r0)
            sage_block(a16b, sb, db, wl_ref[1, layer], bl_ref[1, layer], r0)

    h_fin_a = bufs_a[num_extra % 2]
    h_fin_b = bufs_b[num_extra % 2]

    pooled[:, 0:h_pad] = jnp.dot(pool16, h_fin_a[...],
                                 preferred_element_type=jnp.float32)
    pooled[:, h_pad:2 * h_pad] = jnp.dot(pool16, h_fin_b[...],
                                         preferred_element_type=jnp.float32)

    t1 = jnp.maximum(jnp.dot(pooled[...], hw1_ref[...],
                             preferred_element_type=jnp.float32)
                     + hb1_ref[...], 0.0)
    t2 = jnp.maximum(jnp.dot(t1, hw2_ref[...],
                             preferred_element_type=jnp.float32)
                     + hb2_ref[...], 0.0)
    logits = jnp.dot(t2, hw3_ref[...],
                     preferred_element_type=jnp.float32) + hb3_ref[...]
    col = jax.lax.broadcasted_iota(jnp.int32, logits.shape, 1)
    logits = jnp.where(col < _NUM_CLASSES_OUT, logits, -1e30)
    m = jnp.max(logits, axis=-1, keepdims=True)
    z = logits - m
    lse = jnp.log(jnp.sum(jnp.exp(z), axis=-1, keepdims=True))
    o_ref[...] = (z - lse)[:, :_NUM_CLASSES_OUT]


def kernel(w1, b1, wl, bl, head_w1, head_b1, head_w2, head_b2,
           head_w3, head_b3, sc_x, sc_adj, fc_x, fc_adj, pool_mat):
    n = sc_x.shape[0]
    g = pool_mat.shape[0]
    h_pad = w1.shape[2]

    tile = next(c for c in (256, 128, 64, 32, 16, 8, 1) if n % c == 0)
    ring = min(_RING, n // tile)

    vmem = pl.BlockSpec(memory_space=pltpu.MemorySpace.VMEM)
    hbm = pl.BlockSpec(memory_space=pl.ANY)

    out = pl.pallas_call(
        _fused_body,
        out_shape=jax.ShapeDtypeStruct((g, _NUM_CLASSES_OUT), jnp.float32),
        in_specs=[vmem, vmem, vmem, vmem,
                  vmem, vmem, vmem, vmem, vmem, vmem,
                  vmem, hbm, vmem, hbm, vmem],
        out_specs=vmem,
        scratch_shapes=[
            pltpu.VMEM((ring, tile, n), jnp.float32),
            pltpu.VMEM((ring, tile, n), jnp.float32),
            pltpu.VMEM((n, n), jnp.bfloat16),
            pltpu.VMEM((n, n), jnp.bfloat16),
            pltpu.SemaphoreType.DMA((2, n // tile)),
            pltpu.VMEM((n, h_pad), jnp.bfloat16),
            pltpu.VMEM((n, h_pad), jnp.bfloat16),
            pltpu.VMEM((n, h_pad), jnp.bfloat16),
            pltpu.VMEM((n, h_pad), jnp.bfloat16),
            pltpu.VMEM((g, 2 * h_pad), jnp.float32),
        ],
        name="graphsage_fused",
    )(w1, b1, wl, bl, head_w1, head_b1, head_w2, head_b2, head_w3, head_b3,
      sc_x, sc_adj, fc_x, fc_adj, pool_mat)
    return out


# speedup vs baseline: 1.0948x; 1.0948x over previous
import jax
import jax.numpy as jnp
from jax.experimental import pallas as pl
from jax.experimental.pallas import tpu as pltpu

_NUM_CLASSES_OUT = 64


def _fused_body(w1_ref, b1_ref, wl_ref, bl_ref,
                hw1_ref, hb1_ref, hw2_ref, hb2_ref, hw3_ref, hb3_ref,
                scx_ref, scadj_hbm, fcx_ref, fcadj_hbm, pool_ref,
                o_ref, abuf_a, abuf_b, a16, sem, h16, xr, pooled):
    n = abuf_a.shape[0]
    nt = sem.shape[1]
    tile = n // nt
    f_pad = w1_ref.shape[1] // 2
    h_pad = w1_ref.shape[2]
    num_extra = wl_ref.shape[1]

    def copy(adj_hbm, abuf, bidx, t):
        return pltpu.make_async_copy(
            adj_hbm.at[pl.ds(t * tile, tile)],
            abuf.at[pl.ds(t * tile, tile)],
            sem.at[bidx, t])

    for t in range(nt):
        copy(scadj_hbm, abuf_a, 0, t).start()
    for t in range(nt):
        copy(fcadj_hbm, abuf_b, 1, t).start()

    pool16 = pool_ref[...].astype(jnp.bfloat16)
    x16a = scx_ref[...].astype(jnp.bfloat16)
    x16b = fcx_ref[...].astype(jnp.bfloat16)

    xr[:, 0:h_pad] = jnp.dot(x16a, w1_ref[0, f_pad:],
                             preferred_element_type=jnp.float32) + b1_ref[0]
    xr[:, h_pad:2 * h_pad] = jnp.dot(x16b, w1_ref[1, f_pad:],
                                     preferred_element_type=jnp.float32) + b1_ref[1]

    def run_branch(adj_hbm, abuf, x16, bidx):
        w1l = w1_ref[bidx, :f_pad]

        for t in range(nt):
            copy(adj_hbm, abuf, bidx, t).wait()
            a_t = abuf[pl.ds(t * tile, tile), :].astype(jnp.bfloat16)
            a16[pl.ds(t * tile, tile), :] = a_t
            agg = jnp.dot(a_t, x16, preferred_element_type=jnp.float32)
            z = (jnp.dot(agg.astype(jnp.bfloat16), w1l,
                         preferred_element_type=jnp.float32)
                 + xr[pl.ds(t * tile, tile),
                      bidx * h_pad:(bidx + 1) * h_pad])
            h16[pl.ds(t * tile, tile), :] = jnp.maximum(z, 0.0).astype(jnp.bfloat16)

        for layer in range(num_extra):
            wlb = wl_ref[bidx, layer]
            blb = bl_ref[bidx, layer]
            agg = jnp.dot(a16[...], h16[...],
                          preferred_element_type=jnp.float32)
            z = (jnp.dot(agg.astype(jnp.bfloat16), wlb[:h_pad],
                         preferred_element_type=jnp.float32)
                 + jnp.dot(h16[...], wlb[h_pad:],
                           preferred_element_type=jnp.float32)
                 + blb)
            h16[...] = jnp.maximum(z, 0.0).astype(jnp.bfloat16)

        pooled[:, bidx * h_pad:(bidx + 1) * h_pad] = jnp.dot(
            pool16, h16[...], preferred_element_type=jnp.float32)

    run_branch(scadj_hbm, abuf_a, x16a, 0)
    run_branch(fcadj_hbm, abuf_b, x16b, 1)

    t1 = jnp.maximum(jnp.dot(pooled[...], hw1_ref[...],
                             preferred_element_type=jnp.float32)
                     + hb1_ref[...], 0.0)
    t2 = jnp.maximum(jnp.dot(t1, hw2_ref[...],
                             preferred_element_type=jnp.float32)
                     + hb2_ref[...], 0.0)
    logits = jnp.dot(t2, hw3_ref[...],
                     preferred_element_type=jnp.float32) + hb3_ref[...]
    col = jax.lax.broadcasted_iota(jnp.int32, logits.shape, 1)
    logits = jnp.where(col < _NUM_CLASSES_OUT, logits, -1e30)
    m = jnp.max(logits, axis=-1, keepdims=True)
    z = logits - m
    lse = jnp.log(jnp.sum(jnp.exp(z), axis=-1, keepdims=True))
    o_ref[...] = (z - lse)[:, :_NUM_CLASSES_OUT]


def kernel(w1, b1, wl, bl, head_w1, head_b1, head_w2, head_b2,
           head_w3, head_b3, sc_x, sc_adj, fc_x, fc_adj, pool_mat):
    n = sc_x.shape[0]
    g = pool_mat.shape[0]
    h_pad = w1.shape[2]

    tile = next(c for c in (512, 256, 128, 64, 32, 16, 8, 1) if n % c == 0)
    tile = min(tile, n)

    vmem = pl.BlockSpec(memory_space=pltpu.MemorySpace.VMEM)
    hbm = pl.BlockSpec(memory_space=pl.ANY)

    out = pl.pallas_call(
        _fused_body,
        out_shape=jax.ShapeDtypeStruct((g, _NUM_CLASSES_OUT), jnp.float32),
        in_specs=[vmem, vmem, vmem, vmem,
                  vmem, vmem, vmem, vmem, vmem, vmem,
                  vmem, hbm, vmem, hbm, vmem],
        out_specs=vmem,
        scratch_shapes=[
            pltpu.VMEM((n, n), jnp.float32),
            pltpu.VMEM((n, n), jnp.float32),
            pltpu.VMEM((n, n), jnp.bfloat16),
            pltpu.SemaphoreType.DMA((2, n // tile)),
            pltpu.VMEM((n, h_pad), jnp.bfloat16),
            pltpu.VMEM((n, 2 * h_pad), jnp.float32),
            pltpu.VMEM((g, 2 * h_pad), jnp.float32),
        ],
        name="graphsage_fused",
    )(w1, b1, wl, bl, head_w1, head_b1, head_w2, head_b2, head_w3, head_b3,
      sc_x, sc_adj, fc_x, fc_adj, pool_mat)
    return out
